# Initial kernel scaffold; baseline (speedup 1.0000x reference)
import jax
import jax.numpy as jnp
from jax.experimental import pallas as pl


def kernel(x, edge_index, W1, b1, Wc, bc, gamma, beta, W2, b2, Wo, bo):
    raise NotImplementedError("write your pallas kernel here")


# speedup vs baseline: 19.0598x; 19.0598x over previous
import functools

import jax
import jax.numpy as jnp
from jax import lax
from jax.experimental import pallas as pl
from jax.experimental.pallas import tpu as pltpu
from jax.experimental.pallas import tpu_sc as plsc

N = 10000
E = 320000
D = 128
H = 128

NC = 2
NS = 16
NW = NC * NS
EPT = E // NW
CH = 80
NCHUNK = EPT // CH
NP = 10240
RPT = NP // NS

@functools.lru_cache(maxsize=None)
def _sc_kernels():
    mesh = plsc.VectorSubcoreMesh(
        core_axis_name="c", subcore_axis_name="s",
        num_cores=NC, num_subcores=NS,
    )

    @functools.partial(
        pl.kernel,
        out_type=jax.ShapeDtypeStruct((NC, NP, 16), jnp.float32),
        mesh=mesh,
        scratch_types=[
            pltpu.VMEM((NCHUNK, CH), jnp.int32),
            pltpu.VMEM((CH, 16), jnp.float32),
            pltpu.VMEM_SHARED((NP, 16), jnp.float32),
        ],
    )
    def sc_hist(dst_hbm, zeros_hbm, out_hbm, dst_v, ones_v, acc_sh):
        c = lax.axis_index("c")
        s = lax.axis_index("s")
        wid = c * NS + s

        pltpu.sync_copy(zeros_hbm.at[pl.ds(s * RPT, RPT)],
                        acc_sh.at[pl.ds(s * RPT, RPT)])
        pltpu.sync_copy(dst_hbm.at[wid], dst_v)

        @pl.loop(0, CH)
        def _(i):
            ones_v[i, :] = jnp.ones((16,), jnp.float32)

        plsc.subcore_barrier()

        @pl.loop(0, NCHUNK)
        def _(j):
            pltpu.sync_copy(ones_v, acc_sh.at[dst_v.at[j]], add=True)

        plsc.subcore_barrier()
        pltpu.sync_copy(acc_sh.at[pl.ds(s * RPT, RPT)],
                        out_hbm.at[c, pl.ds(s * RPT, RPT)])

    @functools.partial(
        pl.kernel,
        out_type=jax.ShapeDtypeStruct((NC, NP, H), jnp.float32),
        mesh=mesh,
        scratch_types=[
            pltpu.VMEM((NCHUNK, CH), jnp.int32),
            pltpu.VMEM((NCHUNK, CH), jnp.int32),
            pltpu.VMEM((CH, H), jnp.float32),
            pltpu.VMEM_SHARED((NP, H), jnp.float32),
        ],
    )
    def sc_conv(g_hbm, src_hbm, dst_hbm, zeros_hbm, out_hbm,
                src_v, dst_v, rows_v, acc_sh):
        c = lax.axis_index("c")
        s = lax.axis_index("s")
        wid = c * NS + s

        pltpu.sync_copy(zeros_hbm.at[pl.ds(s * RPT, RPT)],
                        acc_sh.at[pl.ds(s * RPT, RPT)])
        pltpu.sync_copy(src_hbm.at[wid], src_v)
        pltpu.sync_copy(dst_hbm.at[wid], dst_v)
        plsc.subcore_barrier()

        @pl.loop(0, NCHUNK)
        def _(j):
            pltpu.sync_copy(g_hbm.at[src_v.at[j]], rows_v)
            pltpu.sync_copy(rows_v, acc_sh.at[dst_v.at[j]], add=True)

        plsc.subcore_barrier()
        pltpu.sync_copy(acc_sh.at[pl.ds(s * RPT, RPT)],
                        out_hbm.at[c, pl.ds(s * RPT, RPT)])

    return sc_hist, sc_conv


def _tc_mm1_body(x_ref, w1_ref, b1_ref, wc_ref, o_ref):
    w1c = jnp.dot(w1_ref[...], wc_ref[...], preferred_element_type=jnp.float32)
    b1c = jnp.dot(b1_ref[...], wc_ref[...], preferred_element_type=jnp.float32)
    o_ref[...] = (
        jnp.dot(x_ref[...], w1c, preferred_element_type=jnp.float32) + b1c
    )


def _tc_g1_body(hist_ref, hc1_ref, g1_ref, dinv_ref):
    deg = hist_ref[0, :N, :1] + hist_ref[1, :N, :1] + 1.0
    dinv = lax.rsqrt(deg)
    dinv_ref[...] = dinv
    g1_ref[...] = hc1_ref[...] * dinv


def _bn_act(pre, gamma, beta):
    mean = jnp.mean(pre, axis=0, keepdims=True)
    var = jnp.mean((pre - mean) ** 2, axis=0, keepdims=True)
    h = (pre - mean) * lax.rsqrt(var + 1e-5) * gamma + beta
    return jnp.where(h > 0, h, 0.01 * h)


def _tc_mid_body(sp_ref, g_ref, dinv_ref, bc_ref, gamma_ref, beta_ref,
                 wc_ref, o_ref):
    dinv = dinv_ref[...]
    pre = dinv * (sp_ref[0, :N] + sp_ref[1, :N] + g_ref[...]) + bc_ref[...]
    a = _bn_act(pre, gamma_ref[...], beta_ref[...])
    o_ref[...] = (
        jnp.dot(a, wc_ref[...], preferred_element_type=jnp.float32) * dinv
    )


def _tc_final_body(sp_ref, g_ref, dinv_ref, bc_ref, gamma_ref, beta_ref,
                   w2_ref, b2_ref, wo_ref, bo_ref, o_ref):
    pre = dinv_ref[...] * (sp_ref[0, :N] + sp_ref[1, :N] + g_ref[...]) + bc_ref[...]
    a = _bn_act(pre, gamma_ref[...], beta_ref[...])
    w2o = jnp.dot(w2_ref[...], wo_ref[...], preferred_element_type=jnp.float32)
    c = jnp.dot(b2_ref[...], wo_ref[...],
                preferred_element_type=jnp.float32) + bo_ref[...]
    z = jnp.dot(a, w2o, preferred_element_type=jnp.float32) + c
    o_ref[...] = jax.nn.sigmoid(z)


def _tc_call(body, out_shape, *args):
    return pl.pallas_call(
        body, out_shape=jax.ShapeDtypeStruct(out_shape, jnp.float32)
    )(*args)


def kernel(x, edge_index, W1, b1, Wc, bc, gamma, beta, W2, b2, Wo, bo):
    src_r = edge_index[0].reshape(NW, NCHUNK, CH)
    dst_r = edge_index[1].reshape(NW, NCHUNK, CH)
    z16 = jnp.zeros((NP, 16), jnp.float32)
    z128 = jnp.zeros((NP, H), jnp.float32)
    b1r = b1.reshape(1, H)
    bcr = bc.reshape(1, H)
    gammar = gamma.reshape(1, H)
    betar = beta.reshape(1, H)
    b2r = b2.reshape(1, H)
    bor = bo.reshape(1, 1)

    sc_hist, sc_conv = _sc_kernels()

    hist = sc_hist(dst_r, z16)
    hc1 = _tc_call(_tc_mm1_body, (N, H), x, W1, b1r, Wc)

    g1, dinv = pl.pallas_call(
        _tc_g1_body,
        out_shape=(
            jax.ShapeDtypeStruct((N, H), jnp.float32),
            jax.ShapeDtypeStruct((N, 1), jnp.float32),
        ),
    )(hist, hc1)

    sp1 = sc_conv(g1, src_r, dst_r, z128)
    g2 = _tc_call(_tc_mid_body, (N, H), sp1, g1, dinv, bcr, gammar, betar, Wc)
    sp2 = sc_conv(g2, src_r, dst_r, z128)
    return _tc_call(_tc_final_body, (N, 1), sp2, g2, dinv, bcr, gammar,
                    betar, W2, b2r, Wo, bor)
